# Initial kernel scaffold; baseline (speedup 1.0000x reference)
import jax
import jax.numpy as jnp
from jax.experimental import pallas as pl


def kernel(x, edge_index, W_self1, W_neigh1, b1, W_self2, W_neigh2, b2):
    raise NotImplementedError("write your pallas kernel here")


# speedup vs baseline: 4.5641x; 4.5641x over previous
import functools

import jax
import jax.numpy as jnp
from jax import lax
from jax.experimental import pallas as pl
from jax.experimental.pallas import tpu as pltpu
from jax.experimental.pallas import tpu_sc as plsc

N = 10000
E = 320000
D_IN = 128
D_HID = 256
HW = 64

NC = 2
NS = 16
CH = 80
NP = 10240
ROWS_PER_TILE = NP // NS
NCHUNK = E // NS // CH
DW = 16
VSL = 16


def _sc_agg1_body(x2_hbm, src_hbm, dst_hbm, zf_hbm, zd_hbm, ones_hbm,
                  aggp_hbm, degp_hbm,
                  idx_s, idx_d, rows, ones, accum, degacc, sem):
    cid = lax.axis_index("c")
    sid = lax.axis_index("s")
    r0 = sid * ROWS_PER_TILE
    pltpu.sync_copy(zf_hbm, accum.at[pl.ds(r0, ROWS_PER_TILE)])
    pltpu.sync_copy(zd_hbm, degacc.at[pl.ds(r0, ROWS_PER_TILE)])
    pltpu.sync_copy(src_hbm.at[sid], idx_s)
    pltpu.sync_copy(dst_hbm.at[sid], idx_d)
    pltpu.sync_copy(ones_hbm, ones)
    two = jnp.full((VSL,), 2, jnp.int32)
    offc = jnp.full((VSL,), cid, jnp.int32)
    plsc.subcore_barrier()

    @pl.loop(0, NCHUNK)
    def _chunk(j):
        for k in range(CH // VSL):
            sl = idx_s[j, pl.ds(k * VSL, VSL)]
            idx_s[j, pl.ds(k * VSL, VSL)] = sl * two + offc
        pltpu.async_copy(x2_hbm.at[idx_s.at[j]], rows, sem).wait()
        pltpu.sync_copy(rows, accum.at[idx_d.at[j]], add=True)

        @pl.when(cid == 0)
        def _():
            pltpu.sync_copy(ones, degacc.at[idx_d.at[j]], add=True)

    plsc.subcore_barrier()
    pltpu.sync_copy(accum.at[pl.ds(r0, ROWS_PER_TILE)],
                    aggp_hbm.at[cid, pl.ds(r0, ROWS_PER_TILE)])

    @pl.when(cid == 0)
    def _():
        pltpu.sync_copy(degacc.at[pl.ds(r0, ROWS_PER_TILE)],
                        degp_hbm.at[pl.ds(r0, ROWS_PER_TILE)])


@functools.cache
def _get_sc_agg1():
    mesh = plsc.VectorSubcoreMesh(core_axis_name="c", subcore_axis_name="s",
                                  num_cores=NC, num_subcores=NS)
    return pl.kernel(
        _sc_agg1_body,
        out_type=(jax.ShapeDtypeStruct((NC, NP, HW), jnp.float32),
                  jax.ShapeDtypeStruct((NP, DW), jnp.float32)),
        mesh=mesh,
        compiler_params=pltpu.CompilerParams(use_tc_tiling_on_sc=False),
        scratch_types=[
            pltpu.VMEM((NCHUNK, CH), jnp.int32),
            pltpu.VMEM((NCHUNK, CH), jnp.int32),
            pltpu.VMEM((CH, HW), jnp.float32),
            pltpu.VMEM((CH, DW), jnp.float32),
            pltpu.VMEM_SHARED((NP, HW), jnp.float32),
            pltpu.VMEM_SHARED((NP, DW), jnp.float32),
            pltpu.SemaphoreType.DMA,
        ],
    )


def _sc_agg2_body(h4_hbm, src_hbm, dst_hbm, zf_hbm,
                  agg2p_hbm,
                  idx_s, idx_d, rows, accum, sem):
    cid = lax.axis_index("c")
    sid = lax.axis_index("s")
    r0 = sid * ROWS_PER_TILE
    pltpu.sync_copy(src_hbm.at[sid], idx_s)
    pltpu.sync_copy(dst_hbm.at[sid], idx_d)
    four = jnp.full((VSL,), 4, jnp.int32)
    offc = jnp.full((VSL,), cid * 2, jnp.int32)
    one = jnp.full((VSL,), 1, jnp.int32)

    for p in range(2):
        pltpu.sync_copy(zf_hbm, accum.at[pl.ds(r0, ROWS_PER_TILE)])
        plsc.subcore_barrier()

        @pl.loop(0, NCHUNK)
        def _chunk(j):
            for k in range(CH // VSL):
                sl = idx_s[j, pl.ds(k * VSL, VSL)]
                if p == 0:
                    idx_s[j, pl.ds(k * VSL, VSL)] = sl * four + offc
                else:
                    idx_s[j, pl.ds(k * VSL, VSL)] = sl + one
            pltpu.async_copy(h4_hbm.at[idx_s.at[j]], rows, sem).wait()
            pltpu.sync_copy(rows, accum.at[idx_d.at[j]], add=True)

        plsc.subcore_barrier()
        pltpu.sync_copy(accum.at[pl.ds(r0, ROWS_PER_TILE)],
                        agg2p_hbm.at[cid * 2 + p, pl.ds(r0, ROWS_PER_TILE)])
        plsc.subcore_barrier()


@functools.cache
def _get_sc_agg2():
    mesh = plsc.VectorSubcoreMesh(core_axis_name="c", subcore_axis_name="s",
                                  num_cores=NC, num_subcores=NS)
    return pl.kernel(
        _sc_agg2_body,
        out_type=jax.ShapeDtypeStruct((4, NP, HW), jnp.float32),
        mesh=mesh,
        compiler_params=pltpu.CompilerParams(use_tc_tiling_on_sc=False),
        scratch_types=[
            pltpu.VMEM((NCHUNK, CH), jnp.int32),
            pltpu.VMEM((NCHUNK, CH), jnp.int32),
            pltpu.VMEM((CH, HW), jnp.float32),
            pltpu.VMEM_SHARED((NP, HW), jnp.float32),
            pltpu.SemaphoreType.DMA,
        ],
    )


R_BLK = 1000


def _tc_layer1_body(x_ref, aggp_ref, degp_ref, ws_ref, wn_ref, b_ref, h_ref):
    deg = degp_ref[:, 0:1]
    inv = 1.0 / jnp.maximum(deg, 1.0)
    nbar = jnp.concatenate([aggp_ref[0], aggp_ref[1]], axis=1) * inv
    h = (jnp.dot(x_ref[...], ws_ref[...], preferred_element_type=jnp.float32)
         + jnp.dot(nbar, wn_ref[...], preferred_element_type=jnp.float32)
         + b_ref[...])
    h_ref[...] = jnp.maximum(h, 0.0)


def _tc_layer2_body(h_ref, agg2p_ref, degp_ref, ws_ref, wn_ref, b_ref, o_ref):
    deg = degp_ref[:, 0:1]
    inv = 1.0 / jnp.maximum(deg, 1.0)
    nbar = jnp.concatenate([agg2p_ref[q] for q in range(4)], axis=1) * inv
    o = (jnp.dot(h_ref[...], ws_ref[...], preferred_element_type=jnp.float32)
         + jnp.dot(nbar, wn_ref[...], preferred_element_type=jnp.float32)
         + b_ref[...])
    o_ref[...] = jnp.maximum(o, 0.0)


def _row_spec(shape_tail):
    return pl.BlockSpec((R_BLK,) + shape_tail, lambda i: (i,) + (0,) * len(shape_tail))


def _part_spec(nparts):
    return pl.BlockSpec((nparts, R_BLK, HW), lambda i: (0, i, 0))


def _full_spec(shape):
    return pl.BlockSpec(shape, lambda i: (0,) * len(shape))


_tc_layer1 = pl.pallas_call(
    _tc_layer1_body,
    grid=(N // R_BLK,),
    in_specs=[
        _row_spec((D_IN,)),
        _part_spec(NC),
        _row_spec((DW,)),
        _full_spec((D_IN, D_HID)),
        _full_spec((D_IN, D_HID)),
        _full_spec((1, D_HID)),
    ],
    out_specs=_row_spec((D_HID,)),
    out_shape=jax.ShapeDtypeStruct((N, D_HID), jnp.float32),
)

_tc_layer2 = pl.pallas_call(
    _tc_layer2_body,
    grid=(N // R_BLK,),
    in_specs=[
        _row_spec((D_HID,)),
        _part_spec(4),
        _row_spec((DW,)),
        _full_spec((D_HID, D_HID)),
        _full_spec((D_HID, D_HID)),
        _full_spec((1, D_HID)),
    ],
    out_specs=_row_spec((D_HID,)),
    out_shape=jax.ShapeDtypeStruct((N, D_HID), jnp.float32),
)


def kernel(x, edge_index, W_self1, W_neigh1, b1, W_self2, W_neigh2, b2):
    src = edge_index[0].astype(jnp.int32).reshape(NS, NCHUNK, CH)
    dst = edge_index[1].astype(jnp.int32).reshape(NS, NCHUNK, CH)
    zf = jnp.zeros((ROWS_PER_TILE, HW), jnp.float32)
    zd = jnp.zeros((ROWS_PER_TILE, DW), jnp.float32)
    ones = jnp.ones((CH, DW), jnp.float32)

    aggp, degp = _get_sc_agg1()(x.reshape(NC * N, HW), src, dst, zf, zd, ones)
    h1 = _tc_layer1(x, aggp, degp, W_self1, W_neigh1, b1.reshape(1, D_HID))
    agg2p = _get_sc_agg2()(h1.reshape(4 * N, HW), src, dst, zf)
    out = _tc_layer2(h1, agg2p, degp, W_self2, W_neigh2, b2.reshape(1, D_HID))
    return out


# speedup vs baseline: 7.7891x; 1.7066x over previous
import functools

import jax
import jax.numpy as jnp
from jax import lax
from jax.experimental import pallas as pl
from jax.experimental.pallas import tpu as pltpu
from jax.experimental.pallas import tpu_sc as plsc

N = 10000
E = 320000
D_IN = 128
D_HID = 256
HW = 64

NC = 2
NS = 16
CH = 80
NP = 10240
ROWS_PER_TILE = NP // NS
NCHUNK = E // NS // CH
DW = 16
VSL = 16


def _sc_agg1_body(x2_hbm, src_hbm, dst_hbm, zf_hbm, zd_hbm, ones_hbm,
                  aggp_hbm, degp_hbm,
                  idx_s, idx_d, rows_a, rows_b, ones, accum, degacc,
                  sem_a, sem_b):
    cid = lax.axis_index("c")
    sid = lax.axis_index("s")
    r0 = sid * ROWS_PER_TILE
    pltpu.sync_copy(zf_hbm, accum.at[pl.ds(r0, ROWS_PER_TILE)])
    pltpu.sync_copy(zd_hbm, degacc.at[pl.ds(r0, ROWS_PER_TILE)])
    pltpu.sync_copy(src_hbm.at[sid], idx_s)
    pltpu.sync_copy(dst_hbm.at[sid], idx_d)
    pltpu.sync_copy(ones_hbm, ones)
    two = jnp.full((VSL,), 2, jnp.int32)
    offc = jnp.full((VSL,), cid, jnp.int32)

    def transform(j):
        for k in range(CH // VSL):
            sl = idx_s[j, pl.ds(k * VSL, VSL)]
            idx_s[j, pl.ds(k * VSL, VSL)] = sl * two + offc

    def gather(j, buf, sem):
        pltpu.async_copy(x2_hbm.at[idx_s.at[j]], buf, sem)

    def wait_gather(j, buf, sem):
        pltpu.make_async_copy(x2_hbm.at[idx_s.at[j]], buf, sem).wait()

    def scatter(j, buf):
        pltpu.sync_copy(buf, accum.at[idx_d.at[j]], add=True)

        @pl.when(cid == 0)
        def _():
            pltpu.sync_copy(ones, degacc.at[idx_d.at[j]], add=True)

    plsc.subcore_barrier()
    transform(0)
    gather(0, rows_a, sem_a)

    @pl.loop(0, NCHUNK, step=2)
    def _pair(j):
        transform(j + 1)
        gather(j + 1, rows_b, sem_b)
        wait_gather(j, rows_a, sem_a)
        scatter(j, rows_a)

        @pl.when(j + 2 < NCHUNK)
        def _():
            transform(j + 2)
            gather(j + 2, rows_a, sem_a)

        wait_gather(j + 1, rows_b, sem_b)
        scatter(j + 1, rows_b)

    plsc.subcore_barrier()
    pltpu.sync_copy(accum.at[pl.ds(r0, ROWS_PER_TILE)],
                    aggp_hbm.at[cid, pl.ds(r0, ROWS_PER_TILE)])

    @pl.when(cid == 0)
    def _():
        pltpu.sync_copy(degacc.at[pl.ds(r0, ROWS_PER_TILE)],
                        degp_hbm.at[pl.ds(r0, ROWS_PER_TILE)])


@functools.cache
def _get_sc_agg1():
    mesh = plsc.VectorSubcoreMesh(core_axis_name="c", subcore_axis_name="s",
                                  num_cores=NC, num_subcores=NS)
    return pl.kernel(
        _sc_agg1_body,
        out_type=(jax.ShapeDtypeStruct((NC, NP, HW), jnp.float32),
                  jax.ShapeDtypeStruct((NP, DW), jnp.float32)),
        mesh=mesh,
        compiler_params=pltpu.CompilerParams(use_tc_tiling_on_sc=False),
        scratch_types=[
            pltpu.VMEM((NCHUNK, CH), jnp.int32),
            pltpu.VMEM((NCHUNK, CH), jnp.int32),
            pltpu.VMEM((CH, HW), jnp.float32),
            pltpu.VMEM((CH, HW), jnp.float32),
            pltpu.VMEM((CH, DW), jnp.float32),
            pltpu.VMEM_SHARED((NP, HW), jnp.float32),
            pltpu.VMEM_SHARED((NP, DW), jnp.float32),
            pltpu.SemaphoreType.DMA,
            pltpu.SemaphoreType.DMA,
        ],
    )


def _sc_agg2_body(h4_hbm, src_hbm, dst_hbm, zf_hbm,
                  agg2p_hbm,
                  idx_s, idx_d, rows_a, rows_b, accum, sem_a, sem_b):
    cid = lax.axis_index("c")
    sid = lax.axis_index("s")
    r0 = sid * ROWS_PER_TILE
    pltpu.sync_copy(src_hbm.at[sid], idx_s)
    pltpu.sync_copy(dst_hbm.at[sid], idx_d)
    four = jnp.full((VSL,), 4, jnp.int32)
    offc = jnp.full((VSL,), cid * 2, jnp.int32)
    one = jnp.full((VSL,), 1, jnp.int32)

    for p in range(2):
        def transform(j, _p=p):
            for k in range(CH // VSL):
                sl = idx_s[j, pl.ds(k * VSL, VSL)]
                if _p == 0:
                    idx_s[j, pl.ds(k * VSL, VSL)] = sl * four + offc
                else:
                    idx_s[j, pl.ds(k * VSL, VSL)] = sl + one

        def gather(j, buf, sem):
            pltpu.async_copy(h4_hbm.at[idx_s.at[j]], buf, sem)

        def wait_gather(j, buf, sem):
            pltpu.make_async_copy(h4_hbm.at[idx_s.at[j]], buf, sem).wait()

        def scatter(j, buf):
            pltpu.sync_copy(buf, accum.at[idx_d.at[j]], add=True)

        pltpu.sync_copy(zf_hbm, accum.at[pl.ds(r0, ROWS_PER_TILE)])
        plsc.subcore_barrier()
        transform(0)
        gather(0, rows_a, sem_a)

        @pl.loop(0, NCHUNK, step=2)
        def _pair(j):
            transform(j + 1)
            gather(j + 1, rows_b, sem_b)
            wait_gather(j, rows_a, sem_a)
            scatter(j, rows_a)

            @pl.when(j + 2 < NCHUNK)
            def _():
                transform(j + 2)
                gather(j + 2, rows_a, sem_a)

            wait_gather(j + 1, rows_b, sem_b)
            scatter(j + 1, rows_b)

        plsc.subcore_barrier()
        pltpu.sync_copy(accum.at[pl.ds(r0, ROWS_PER_TILE)],
                        agg2p_hbm.at[cid * 2 + p, pl.ds(r0, ROWS_PER_TILE)])
        plsc.subcore_barrier()


@functools.cache
def _get_sc_agg2():
    mesh = plsc.VectorSubcoreMesh(core_axis_name="c", subcore_axis_name="s",
                                  num_cores=NC, num_subcores=NS)
    return pl.kernel(
        _sc_agg2_body,
        out_type=jax.ShapeDtypeStruct((4, NP, HW), jnp.float32),
        mesh=mesh,
        compiler_params=pltpu.CompilerParams(use_tc_tiling_on_sc=False),
        scratch_types=[
            pltpu.VMEM((NCHUNK, CH), jnp.int32),
            pltpu.VMEM((NCHUNK, CH), jnp.int32),
            pltpu.VMEM((CH, HW), jnp.float32),
            pltpu.VMEM((CH, HW), jnp.float32),
            pltpu.VMEM_SHARED((NP, HW), jnp.float32),
            pltpu.SemaphoreType.DMA,
            pltpu.SemaphoreType.DMA,
        ],
    )


R_BLK = 1000


def _tc_layer1_body(x_ref, aggp_ref, degp_ref, ws_ref, wn_ref, b_ref, h_ref):
    deg = degp_ref[:, 0:1]
    inv = 1.0 / jnp.maximum(deg, 1.0)
    nbar = jnp.concatenate([aggp_ref[0], aggp_ref[1]], axis=1) * inv
    h = (jnp.dot(x_ref[...], ws_ref[...], preferred_element_type=jnp.float32)
         + jnp.dot(nbar, wn_ref[...], preferred_element_type=jnp.float32)
         + b_ref[...])
    h_ref[...] = jnp.maximum(h, 0.0)


def _tc_layer2_body(h_ref, agg2p_ref, degp_ref, ws_ref, wn_ref, b_ref, o_ref):
    deg = degp_ref[:, 0:1]
    inv = 1.0 / jnp.maximum(deg, 1.0)
    nbar = jnp.concatenate([agg2p_ref[q] for q in range(4)], axis=1) * inv
    o = (jnp.dot(h_ref[...], ws_ref[...], preferred_element_type=jnp.float32)
         + jnp.dot(nbar, wn_ref[...], preferred_element_type=jnp.float32)
         + b_ref[...])
    o_ref[...] = jnp.maximum(o, 0.0)


def _row_spec(shape_tail):
    return pl.BlockSpec((R_BLK,) + shape_tail, lambda i: (i,) + (0,) * len(shape_tail))


def _part_spec(nparts):
    return pl.BlockSpec((nparts, R_BLK, HW), lambda i: (0, i, 0))


def _full_spec(shape):
    return pl.BlockSpec(shape, lambda i: (0,) * len(shape))


_tc_layer1 = pl.pallas_call(
    _tc_layer1_body,
    grid=(N // R_BLK,),
    in_specs=[
        _row_spec((D_IN,)),
        _part_spec(NC),
        _row_spec((DW,)),
        _full_spec((D_IN, D_HID)),
        _full_spec((D_IN, D_HID)),
        _full_spec((1, D_HID)),
    ],
    out_specs=_row_spec((D_HID,)),
    out_shape=jax.ShapeDtypeStruct((N, D_HID), jnp.float32),
)

_tc_layer2 = pl.pallas_call(
    _tc_layer2_body,
    grid=(N // R_BLK,),
    in_specs=[
        _row_spec((D_HID,)),
        _part_spec(4),
        _row_spec((DW,)),
        _full_spec((D_HID, D_HID)),
        _full_spec((D_HID, D_HID)),
        _full_spec((1, D_HID)),
    ],
    out_specs=_row_spec((D_HID,)),
    out_shape=jax.ShapeDtypeStruct((N, D_HID), jnp.float32),
)


def kernel(x, edge_index, W_self1, W_neigh1, b1, W_self2, W_neigh2, b2):
    src = edge_index[0].astype(jnp.int32).reshape(NS, NCHUNK, CH)
    dst = edge_index[1].astype(jnp.int32).reshape(NS, NCHUNK, CH)
    zf = jnp.zeros((ROWS_PER_TILE, HW), jnp.float32)
    zd = jnp.zeros((ROWS_PER_TILE, DW), jnp.float32)
    ones = jnp.ones((CH, DW), jnp.float32)

    aggp, degp = _get_sc_agg1()(x.reshape(NC * N, HW), src, dst, zf, zd, ones)
    h1 = _tc_layer1(x, aggp, degp, W_self1, W_neigh1, b1.reshape(1, D_HID))
    agg2p = _get_sc_agg2()(h1.reshape(4 * N, HW), src, dst, zf)
    out = _tc_layer2(h1, agg2p, degp, W_self2, W_neigh2, b2.reshape(1, D_HID))
    return out
